# baseline (speedup 1.0000x reference)
import functools

import jax
import jax.numpy as jnp
from jax import lax
from jax.experimental import pallas as pl
from jax.experimental.pallas import tpu as pltpu
from jax.experimental.pallas import tpu_sc as plsc

D_MODEL = 64
NUM_CORES = 2
NUM_SUBCORES = 16
NUM_WORKERS = NUM_CORES * NUM_SUBCORES
CHUNK = 640


@functools.cache
def _build_gather(n_rows: int, vocab: int):
    assert n_rows % (NUM_WORKERS * 8) == 0
    rows_per_worker = n_rows // NUM_WORKERS
    assert rows_per_worker % (2 * CHUNK) == 0
    n_chunks = rows_per_worker // CHUNK
    n_pairs = n_chunks // 2

    mesh = plsc.VectorSubcoreMesh(core_axis_name="c", subcore_axis_name="s")

    @functools.partial(
        pl.kernel,
        mesh=mesh,
        out_type=jax.ShapeDtypeStruct((n_rows, D_MODEL), jnp.float32),
        scratch_types=[
            pltpu.VMEM((rows_per_worker,), jnp.int32),
            pltpu.VMEM((CHUNK, D_MODEL), jnp.float32),
            pltpu.VMEM((CHUNK, D_MODEL), jnp.float32),
            pltpu.SemaphoreType.DMA,
            pltpu.SemaphoreType.DMA,
            pltpu.SemaphoreType.DMA,
            pltpu.SemaphoreType.DMA,
        ],
        compiler_params=pltpu.CompilerParams(use_tc_tiling_on_sc=False),
    )
    def gather_kernel(idx_hbm, table_hbm, out_hbm, idx_v, rows0, rows1,
                      gsem0, gsem1, ssem0, ssem1):
        wid = lax.axis_index("s") * NUM_CORES + lax.axis_index("c")
        base = wid * rows_per_worker
        rows = (rows0, rows1)
        gsem = (gsem0, gsem1)
        ssem = (ssem0, ssem1)

        pltpu.sync_copy(idx_hbm.at[pl.ds(base, rows_per_worker)], idx_v)

        def start_gather(c, b):
            return pltpu.async_copy(
                table_hbm.at[idx_v.at[pl.ds(c * CHUNK, CHUNK)]], rows[b],
                gsem[b])

        def wait_gather(b):
            pltpu.make_async_copy(
                table_hbm.at[idx_v.at[pl.ds(0, CHUNK)]], rows[b],
                gsem[b]).wait()

        def start_store(c, b):
            return pltpu.async_copy(
                rows[b], out_hbm.at[pl.ds(base + c * CHUNK, CHUNK)], ssem[b])

        def wait_store(b):
            pltpu.make_async_copy(
                rows[b], out_hbm.at[pl.ds(base, CHUNK)], ssem[b]).wait()

        start_gather(0, 0)
        start_gather(1, 1)

        def pair_body(p, _):
            c0 = p * 2
            for b in range(2):
                c = c0 + b
                wait_gather(b)
                start_store(c, b)
                wait_store(b)
                start_gather(c + 2, b)
            return 0

        lax.fori_loop(0, n_pairs - 1, pair_body, 0)

        for b in range(2):
            c = n_chunks - 2 + b
            wait_gather(b)
            start_store(c, b)
        for b in range(2):
            wait_store(b)

    return gather_kernel


def kernel(x, table):
    n_rows = x.size
    flat_idx = x.reshape(n_rows).astype(jnp.int32)
    out = _build_gather(n_rows, table.shape[0])(flat_idx, table)
    return out.reshape(x.shape + (D_MODEL,))
